# baseline (speedup 1.0000x reference)
import functools

import jax
import jax.numpy as jnp
from jax import lax
from jax.experimental import pallas as pl
from jax.experimental.pallas import tpu as pltpu
from jax.experimental.pallas import tpu_sc as plsc

NUM_CORES = 2
NUM_SUBCORES = 16
NW = NUM_CORES * NUM_SUBCORES

BATCH = 4096
SEQ = 50
D = 64
B = BATCH * SEQ
CB = 2
CHUNK = CB * SEQ
GCHUNK = 104
PAD = 112
BPW = BATCH // NW
NCHUNK = BPW // CB

RING = 8
AHEAD = 5

_mesh = plsc.VectorSubcoreMesh(core_axis_name="c", subcore_axis_name="s")


@functools.partial(
    pl.kernel,
    out_type=jax.ShapeDtypeStruct((BATCH, SEQ, D), jnp.float32),
    mesh=_mesh,
    scratch_types=[
        pltpu.VMEM((NCHUNK, PAD), jnp.int32),
        pltpu.VMEM((RING, GCHUNK, D), jnp.float32),
        [pltpu.SemaphoreType.DMA] * RING,
        [pltpu.SemaphoreType.DMA] * RING,
    ],
    compiler_params=pltpu.CompilerParams(use_tc_tiling_on_sc=False),
)
def _gather_kernel(table, xpad, out, idx_v, rows_v, gsems, wsems):
    wid = lax.axis_index("s") * NUM_CORES + lax.axis_index("c")
    bbase = wid * BPW
    cbase = wid * NCHUNK

    pltpu.sync_copy(xpad.at[pl.ds(cbase, NCHUNK)], idx_v)

    def gather(j, slot):
        return pltpu.make_async_copy(
            table.at[idx_v.at[j, pl.ds(0, GCHUNK)]], rows_v.at[slot],
            gsems[slot],
        )

    def wb_copies(j, slot):
        b0 = bbase + j * CB
        return [
            pltpu.make_async_copy(
                rows_v.at[slot, pl.ds(k * SEQ, SEQ)], out.at[b0 + k],
                wsems[slot],
            )
            for k in range(CB)
        ]

    wb_waited = 0
    for j in range(min(AHEAD, NCHUNK)):
        gather(j, j % RING).start()
    for j in range(NCHUNK):
        slot = j % RING
        gather(j, slot).wait()
        for c in wb_copies(j, slot):
            c.start()
        nxt = j + AHEAD
        if nxt < NCHUNK:
            prev = nxt - RING
            if prev >= 0:
                for c in wb_copies(prev, prev % RING):
                    c.wait()
                wb_waited = prev + 1
            gather(nxt, nxt % RING).start()
    for j in range(wb_waited, NCHUNK):
        for c in wb_copies(j, j % RING):
            c.wait()


def kernel(x, W_frame_0, W_frame_1):
    table = jnp.concatenate([W_frame_0, W_frame_1], axis=0)
    xpad = jnp.pad(x.reshape(B // CHUNK, CHUNK), ((0, 0), (0, PAD - CHUNK)))
    return _gather_kernel(table, xpad)
